# Initial kernel scaffold; baseline (speedup 1.0000x reference)
import jax
import jax.numpy as jnp
from jax.experimental import pallas as pl


def kernel(positions, edge_index, species, batch, W_embed, W_r1, W_r2, W_msg0, W_vec0, W_up0, W_msg1, W_vec1, W_up1, W_ro1, W_ro2, atomic_E):
    raise NotImplementedError("write your pallas kernel here")


# speedup vs baseline: 21.9673x; 21.9673x over previous
import functools

import jax
import jax.numpy as jnp
from jax import lax
from jax.experimental import pallas as pl
from jax.experimental.pallas import tpu as pltpu
from jax.experimental.pallas import tpu_sc as plsc

NN = 10000
EE = 160000
CC = 64
RBF = 8
NGR = 16
RCUT = 5.0
AVGN = 16.0

NCORE = 2
NSUB = 16

CH = 128
NCHUNK = EE // CH

BE = 1280
BN = 1000

_F32 = jnp.float32


def _dot(a, b):
    return lax.dot_general(a, b, (((1,), (0,)), ((), ())),
                           preferred_element_type=_F32,
                           precision=lax.Precision.HIGHEST)


def _silu(x):
    return x * (1.0 / (1.0 + jnp.exp(-x)))


def _gather0_body(src_hbm, dst_hbm, pos_hbm, hmv_hbm,
                  vt_hbm, hsg_hbm,
                  pos_v, sidx, didx, vbuf, hsb, sem):
    c = lax.axis_index("c")
    s = lax.axis_index("s")
    wid = s * NCORE + c

    pltpu.sync_copy(pos_hbm, pos_v)
    zero16 = jnp.zeros((16,), _F32)
    for j in range(3, 8):
        for g in range(CH // 16):
            vbuf[j, pl.ds(g * 16, 16)] = zero16

    def chunk(i):
        k = i * (NCORE * NSUB) + wid

        @pl.when(k < NCHUNK)
        def _():
            base = k * CH
            pltpu.sync_copy(src_hbm.at[pl.ds(base, CH)], sidx)
            pltpu.sync_copy(dst_hbm.at[pl.ds(base, CH)], didx)
            pltpu.async_copy(hmv_hbm.at[sidx], hsb, sem).wait()
            pltpu.sync_copy(hsb, hsg_hbm.at[pl.ds(base, CH)])
            for g in range(CH // 16):
                sv = sidx[pl.ds(g * 16, 16)] * 8
                dv = didx[pl.ds(g * 16, 16)] * 8
                for j in range(3):
                    ps = plsc.load_gather(pos_v, [sv + j])
                    pd = plsc.load_gather(pos_v, [dv + j])
                    vbuf[j, pl.ds(g * 16, 16)] = pd - ps
            pltpu.sync_copy(vbuf, vt_hbm.at[:, pl.ds(base, CH)])

    pl.loop(0, pl.cdiv(NCHUNK, NCORE * NSUB))(chunk)


def _sc_gather0(src, dst, pos1d, hmv):
    mesh = plsc.VectorSubcoreMesh(core_axis_name="c", subcore_axis_name="s")
    f = functools.partial(
        pl.kernel, _gather0_body, mesh=mesh,
        out_type=(jax.ShapeDtypeStruct((8, EE), _F32),
                  jax.ShapeDtypeStruct((EE, 2 * CC), _F32)),
        scratch_types=[pltpu.VMEM((NN * 8,), _F32),
                       pltpu.VMEM((CH,), jnp.int32),
                       pltpu.VMEM((CH,), jnp.int32),
                       pltpu.VMEM((8, CH), _F32),
                       pltpu.VMEM((CH, 2 * CC), _F32),
                       pltpu.SemaphoreType.DMA],
        compiler_params=pltpu.CompilerParams(needs_layout_passes=False),
    )
    return f()(src, dst, pos1d, hmv)


def _gather1_body(src_hbm, hmv_hbm, hsg_hbm, sidx, hsb, sem):
    c = lax.axis_index("c")
    s = lax.axis_index("s")
    wid = s * NCORE + c

    def chunk(i):
        k = i * (NCORE * NSUB) + wid

        @pl.when(k < NCHUNK)
        def _():
            base = k * CH
            pltpu.sync_copy(src_hbm.at[pl.ds(base, CH)], sidx)
            pltpu.async_copy(hmv_hbm.at[sidx], hsb, sem).wait()
            pltpu.sync_copy(hsb, hsg_hbm.at[pl.ds(base, CH)])

    pl.loop(0, pl.cdiv(NCHUNK, NCORE * NSUB))(chunk)


def _sc_gather1(src, hmv):
    mesh = plsc.VectorSubcoreMesh(core_axis_name="c", subcore_axis_name="s")
    f = functools.partial(
        pl.kernel, _gather1_body, mesh=mesh,
        out_type=jax.ShapeDtypeStruct((EE, 2 * CC), _F32),
        scratch_types=[pltpu.VMEM((CH,), jnp.int32),
                       pltpu.VMEM((CH, 2 * CC), _F32),
                       pltpu.SemaphoreType.DMA],
    )
    return f()(src, hmv)


def _scatter_body(m_hbm, dst_hbm, zeros_hbm, a_hbm, didx, mbuf, acc, sem):
    c = lax.axis_index("c")
    s = lax.axis_index("s")

    @pl.when(s == 0)
    def _():
        pltpu.sync_copy(zeros_hbm, acc)

    plsc.subcore_barrier()

    def chunk(i):
        k = i * NSUB + s

        @pl.when(k < NCHUNK)
        def _():
            base = k * CH
            pltpu.sync_copy(dst_hbm.at[pl.ds(base, CH)], didx)
            pltpu.sync_copy(m_hbm.at[c, pl.ds(base, CH)], mbuf)
            pltpu.sync_copy(mbuf, acc.at[didx], add=True)

    pl.loop(0, pl.cdiv(NCHUNK, NSUB))(chunk)
    plsc.subcore_barrier()

    @pl.when(s == 0)
    def _():
        pltpu.sync_copy(acc, a_hbm.at[c])


def _sc_scatter(m, dst, zeros_acc):
    mesh = plsc.VectorSubcoreMesh(core_axis_name="c", subcore_axis_name="s")
    f = functools.partial(
        pl.kernel, _scatter_body, mesh=mesh,
        out_type=jax.ShapeDtypeStruct((2, NN, 2 * CC), _F32),
        scratch_types=[pltpu.VMEM((CH,), jnp.int32),
                       pltpu.VMEM((CH, 2 * CC), _F32),
                       pltpu.VMEM_SHARED((NN, 2 * CC), _F32),
                       pltpu.SemaphoreType.DMA],
    )
    return f()(m, dst, zeros_acc)


def _node0_body(spec_ref, we_ref, wm_ref, wv_ref, out_ref):
    oh = (spec_ref[...] == lax.broadcasted_iota(jnp.int32, (1, 16), 1))
    h0 = _dot(oh.astype(_F32), we_ref[...])
    out_ref[...] = jnp.concatenate(
        [_dot(h0, wm_ref[...]), _dot(h0, wv_ref[...])], axis=1)


def _tc_node0(spec, we16, wm, wv):
    return pl.pallas_call(
        _node0_body,
        grid=(NN // BN,),
        in_specs=[pl.BlockSpec((BN, 1), lambda i: (i, 0)),
                  pl.BlockSpec((16, CC), lambda i: (0, 0)),
                  pl.BlockSpec((CC, CC), lambda i: (0, 0)),
                  pl.BlockSpec((CC, CC), lambda i: (0, 0))],
        out_specs=pl.BlockSpec((BN, 2 * CC), lambda i: (i, 0)),
        out_shape=jax.ShapeDtypeStruct((NN, 2 * CC), _F32),
    )(spec, we16, wm, wv)


def _geom_body(vt_ref, wr1_ref, wr2_ref, r_ref, u_ref):
    v = vt_ref[...]
    r2 = jnp.sum(v * v, axis=0, keepdims=True) + 1e-12
    r = jnp.sqrt(r2)
    rinv = 1.0 / r
    uT = v * rinv
    eye8 = (lax.broadcasted_iota(jnp.int32, (8, 8), 0)
            == lax.broadcasted_iota(jnp.int32, (8, 8), 1)).astype(_F32)

    def t8(a):
        return lax.dot_general(a, eye8, (((0,), (0,)), ((), ())),
                               preferred_element_type=_F32,
                               precision=lax.Precision.HIGHEST)

    u_ref[...] = t8(uT)
    x = r / RCUT
    nv = lax.broadcasted_iota(jnp.int32, (RBF, 1), 0).astype(_F32) + 1.0
    bes = jnp.sqrt(2.0 / RCUT) * jnp.sin(nv * (jnp.pi * x)) * rinv
    x6 = x * x * x
    x6 = x6 * x6
    fcut = (1.0 - 28.0 * x6 + 48.0 * x6 * x - 21.0 * x6 * x * x)
    fcut = jnp.where(x < 1.0, fcut, 0.0)
    rb = t8(bes * fcut)
    r_ref[...] = _dot(_silu(_dot(rb, wr1_ref[...])), wr2_ref[...])


def _tc_geom(vecT, wr1, wr2):
    return pl.pallas_call(
        _geom_body,
        grid=(EE // BE,),
        in_specs=[pl.BlockSpec((8, BE), lambda i: (0, i)),
                  pl.BlockSpec((RBF, CC), lambda i: (0, 0)),
                  pl.BlockSpec((CC, CC), lambda i: (0, 0))],
        out_specs=[pl.BlockSpec((BE, CC), lambda i: (i, 0)),
                   pl.BlockSpec((BE, 8), lambda i: (i, 0))],
        out_shape=[jax.ShapeDtypeStruct((EE, CC), _F32),
                   jax.ShapeDtypeStruct((EE, 8), _F32)],
    )(vecT, wr1, wr2)


def _msg_body(r_ref, u_ref, hs_ref, m_ref):
    rr = r_ref[...]
    hs = hs_ref[...]
    u = u_ref[...]
    m0 = rr * hs[:, :CC]
    m1 = rr * hs[:, CC:]
    m_ref[0] = jnp.concatenate([m0, m1 * u[:, 0:1]], axis=1)
    m_ref[1] = jnp.concatenate([m1 * u[:, 1:2], m1 * u[:, 2:3]], axis=1)


def _tc_msg(r_arr, u8, hsg):
    return pl.pallas_call(
        _msg_body,
        grid=(EE // BE,),
        in_specs=[pl.BlockSpec((BE, CC), lambda i: (i, 0)),
                  pl.BlockSpec((BE, 8), lambda i: (i, 0)),
                  pl.BlockSpec((BE, 2 * CC), lambda i: (i, 0))],
        out_specs=pl.BlockSpec((2, BE, 2 * CC), lambda i: (0, i, 0)),
        out_shape=jax.ShapeDtypeStruct((2, EE, 2 * CC), _F32),
    )(r_arr, u8, hsg)


def _node_body(a_ref, wu_ref, wm_ref, wv_ref, out_ref):
    a = a_ref[...] * (1.0 / AVGN)
    a1x = a[0, :, CC:]
    a1y = a[1, :, :CC]
    a1z = a[1, :, CC:]
    inv = a[0, :, :CC] + a1x * a1x + a1y * a1y + a1z * a1z
    h = _silu(_dot(inv, wu_ref[...]))
    out_ref[...] = jnp.concatenate(
        [_dot(h, wm_ref[...]), _dot(h, wv_ref[...])], axis=1)


def _tc_node(a_arr, wu, wm, wv):
    return pl.pallas_call(
        _node_body,
        grid=(NN // BN,),
        in_specs=[pl.BlockSpec((2, BN, 2 * CC), lambda i: (0, i, 0)),
                  pl.BlockSpec((CC, CC), lambda i: (0, 0)),
                  pl.BlockSpec((CC, CC), lambda i: (0, 0)),
                  pl.BlockSpec((CC, CC), lambda i: (0, 0))],
        out_specs=pl.BlockSpec((BN, 2 * CC), lambda i: (i, 0)),
        out_shape=jax.ShapeDtypeStruct((NN, 2 * CC), _F32),
    )(a_arr, wu, wm, wv)


def _final_body(a_ref, wu_ref, w1_ref, w2_ref, spec_ref, ae_ref, bat_ref,
                out_ref):
    @pl.when(pl.program_id(0) == 0)
    def _():
        out_ref[...] = jnp.zeros_like(out_ref)

    a = a_ref[...] * (1.0 / AVGN)
    a1x = a[0, :, CC:]
    a1y = a[1, :, :CC]
    a1z = a[1, :, CC:]
    inv = a[0, :, :CC] + a1x * a1x + a1y * a1y + a1z * a1z
    h = _silu(_dot(inv, wu_ref[...]))
    e = _dot(_silu(_dot(h, w1_ref[...])), w2_ref[...])
    oh_s = (spec_ref[...] == lax.broadcasted_iota(jnp.int32, (1, 16), 1))
    e = e + _dot(oh_s.astype(_F32), ae_ref[...])
    oh_b = (bat_ref[...] == lax.broadcasted_iota(jnp.int32, (1, 16), 1))
    out_ref[...] += jnp.sum(oh_b.astype(_F32) * e, axis=0, keepdims=True)


def _tc_final(a_arr, wu, w1, w2, spec, ae16, bat):
    return pl.pallas_call(
        _final_body,
        grid=(NN // BN,),
        in_specs=[pl.BlockSpec((2, BN, 2 * CC), lambda i: (0, i, 0)),
                  pl.BlockSpec((CC, CC), lambda i: (0, 0)),
                  pl.BlockSpec((CC, 16), lambda i: (0, 0)),
                  pl.BlockSpec((16, 1), lambda i: (0, 0)),
                  pl.BlockSpec((BN, 1), lambda i: (i, 0)),
                  pl.BlockSpec((16, 1), lambda i: (0, 0)),
                  pl.BlockSpec((BN, 1), lambda i: (i, 0))],
        out_specs=pl.BlockSpec((1, 16), lambda i: (0, 0)),
        out_shape=jax.ShapeDtypeStruct((1, 16), _F32),
    )(a_arr, wu, w1, w2, spec, ae16, bat)


def kernel(positions, edge_index, species, batch,
           W_embed, W_r1, W_r2,
           W_msg0, W_vec0, W_up0,
           W_msg1, W_vec1, W_up1,
           W_ro1, W_ro2, atomic_E):
    src = edge_index[0].astype(jnp.int32)
    dst = edge_index[1].astype(jnp.int32)
    pos1d = jnp.pad(positions.astype(_F32), ((0, 0), (0, 5))).reshape(-1)
    we16 = jnp.pad(W_embed, ((0, 6), (0, 0)))
    ae16 = jnp.pad(atomic_E, (0, 6)).reshape(16, 1)
    spec = species.astype(jnp.int32).reshape(NN, 1)
    bat = batch.astype(jnp.int32).reshape(NN, 1)
    zeros_acc = jnp.zeros((NN, 2 * CC), _F32)

    hmv = _tc_node0(spec, we16, W_msg0, W_vec0)
    vecT, hsg = _sc_gather0(src, dst, pos1d, hmv)
    r_arr, u8 = _tc_geom(vecT, W_r1, W_r2)

    m = _tc_msg(r_arr, u8, hsg)
    a_arr = _sc_scatter(m, dst, zeros_acc)
    hmv = _tc_node(a_arr, W_up0, W_msg1, W_vec1)

    hsg = _sc_gather1(src, hmv)
    m = _tc_msg(r_arr, u8, hsg)
    a_arr = _sc_scatter(m, dst, zeros_acc)

    energy = _tc_final(a_arr, W_up1, W_ro1, W_ro2, spec, ae16, bat)
    return energy.reshape(NGR)


# speedup vs baseline: 28.7796x; 1.3101x over previous
import functools

import jax
import jax.numpy as jnp
from jax import lax
from jax.experimental import pallas as pl
from jax.experimental.pallas import tpu as pltpu
from jax.experimental.pallas import tpu_sc as plsc

NN = 10000
EE = 160000
CC = 64
RBF = 8
NGR = 16
RCUT = 5.0
AVGN = 16.0

NCORE = 2
NSUB = 16

CH = 128
NCHUNK = EE // CH

BE = 1280
BN = 1000

_F32 = jnp.float32


def _dot(a, b):
    return lax.dot_general(a, b, (((1,), (0,)), ((), ())),
                           preferred_element_type=_F32,
                           precision=lax.Precision.DEFAULT)


def _silu(x):
    return x * (1.0 / (1.0 + jnp.exp(-x)))


def _gather0_body(src_hbm, dst_hbm, pos_hbm, hmv_hbm,
                  vt_hbm, hsg_hbm,
                  pos_v, sidx, didx, vbuf, hsb, sem):
    c = lax.axis_index("c")
    s = lax.axis_index("s")
    wid = s * NCORE + c

    pltpu.sync_copy(pos_hbm, pos_v)
    zero16 = jnp.zeros((16,), _F32)
    for j in range(3, 8):
        for g in range(CH // 16):
            vbuf[j, pl.ds(g * 16, 16)] = zero16

    def chunk(i):
        k = i * (NCORE * NSUB) + wid

        @pl.when(k < NCHUNK)
        def _():
            base = k * CH
            pltpu.sync_copy(src_hbm.at[pl.ds(base, CH)], sidx)
            pltpu.sync_copy(dst_hbm.at[pl.ds(base, CH)], didx)
            pltpu.async_copy(hmv_hbm.at[sidx], hsb, sem).wait()
            pltpu.sync_copy(hsb, hsg_hbm.at[pl.ds(base, CH)])
            for g in range(CH // 16):
                sv = sidx[pl.ds(g * 16, 16)] * 8
                dv = didx[pl.ds(g * 16, 16)] * 8
                for j in range(3):
                    ps = plsc.load_gather(pos_v, [sv + j])
                    pd = plsc.load_gather(pos_v, [dv + j])
                    vbuf[j, pl.ds(g * 16, 16)] = pd - ps
            pltpu.sync_copy(vbuf, vt_hbm.at[:, pl.ds(base, CH)])

    pl.loop(0, pl.cdiv(NCHUNK, NCORE * NSUB))(chunk)


def _sc_gather0(src, dst, pos1d, hmv):
    mesh = plsc.VectorSubcoreMesh(core_axis_name="c", subcore_axis_name="s")
    f = functools.partial(
        pl.kernel, _gather0_body, mesh=mesh,
        out_type=(jax.ShapeDtypeStruct((8, EE), _F32),
                  jax.ShapeDtypeStruct((EE, 2 * CC), _F32)),
        scratch_types=[pltpu.VMEM((NN * 8,), _F32),
                       pltpu.VMEM((CH,), jnp.int32),
                       pltpu.VMEM((CH,), jnp.int32),
                       pltpu.VMEM((8, CH), _F32),
                       pltpu.VMEM((CH, 2 * CC), _F32),
                       pltpu.SemaphoreType.DMA],
        compiler_params=pltpu.CompilerParams(needs_layout_passes=False),
    )
    return f()(src, dst, pos1d, hmv)


def _gather1_body(src_hbm, hmv_hbm, hsg_hbm, sidx, hsb, sem):
    c = lax.axis_index("c")
    s = lax.axis_index("s")
    wid = s * NCORE + c

    def chunk(i):
        k = i * (NCORE * NSUB) + wid

        @pl.when(k < NCHUNK)
        def _():
            base = k * CH
            pltpu.sync_copy(src_hbm.at[pl.ds(base, CH)], sidx)
            pltpu.async_copy(hmv_hbm.at[sidx], hsb, sem).wait()
            pltpu.sync_copy(hsb, hsg_hbm.at[pl.ds(base, CH)])

    pl.loop(0, pl.cdiv(NCHUNK, NCORE * NSUB))(chunk)


def _sc_gather1(src, hmv):
    mesh = plsc.VectorSubcoreMesh(core_axis_name="c", subcore_axis_name="s")
    f = functools.partial(
        pl.kernel, _gather1_body, mesh=mesh,
        out_type=jax.ShapeDtypeStruct((EE, 2 * CC), _F32),
        scratch_types=[pltpu.VMEM((CH,), jnp.int32),
                       pltpu.VMEM((CH, 2 * CC), _F32),
                       pltpu.SemaphoreType.DMA],
    )
    return f()(src, hmv)


def _scatter_body(m_hbm, dst_hbm, zeros_hbm, a_hbm, didx, mbuf, acc, sem):
    c = lax.axis_index("c")
    s = lax.axis_index("s")

    @pl.when(s == 0)
    def _():
        pltpu.sync_copy(zeros_hbm, acc)

    plsc.subcore_barrier()

    def chunk(i):
        k = i * NSUB + s

        @pl.when(k < NCHUNK)
        def _():
            base = k * CH
            pltpu.sync_copy(dst_hbm.at[pl.ds(base, CH)], didx)
            pltpu.sync_copy(m_hbm.at[c, pl.ds(base, CH)], mbuf)
            pltpu.sync_copy(mbuf, acc.at[didx], add=True)

    pl.loop(0, pl.cdiv(NCHUNK, NSUB))(chunk)
    plsc.subcore_barrier()

    @pl.when(s == 0)
    def _():
        pltpu.sync_copy(acc, a_hbm.at[c])


def _sc_scatter(m, dst, zeros_acc):
    mesh = plsc.VectorSubcoreMesh(core_axis_name="c", subcore_axis_name="s")
    f = functools.partial(
        pl.kernel, _scatter_body, mesh=mesh,
        out_type=jax.ShapeDtypeStruct((2, NN, 2 * CC), _F32),
        scratch_types=[pltpu.VMEM((CH,), jnp.int32),
                       pltpu.VMEM((CH, 2 * CC), _F32),
                       pltpu.VMEM_SHARED((NN, 2 * CC), _F32),
                       pltpu.SemaphoreType.DMA],
    )
    return f()(m, dst, zeros_acc)


def _node0_body(spec_ref, we_ref, wm_ref, wv_ref, out_ref):
    oh = (spec_ref[...] == lax.broadcasted_iota(jnp.int32, (1, 16), 1))
    h0 = _dot(oh.astype(_F32), we_ref[...])
    out_ref[...] = jnp.concatenate(
        [_dot(h0, wm_ref[...]), _dot(h0, wv_ref[...])], axis=1)


def _tc_node0(spec, we16, wm, wv):
    return pl.pallas_call(
        _node0_body,
        grid=(NN // BN,),
        in_specs=[pl.BlockSpec((BN, 1), lambda i: (i, 0)),
                  pl.BlockSpec((16, CC), lambda i: (0, 0)),
                  pl.BlockSpec((CC, CC), lambda i: (0, 0)),
                  pl.BlockSpec((CC, CC), lambda i: (0, 0))],
        out_specs=pl.BlockSpec((BN, 2 * CC), lambda i: (i, 0)),
        out_shape=jax.ShapeDtypeStruct((NN, 2 * CC), _F32),
    )(spec, we16, wm, wv)


def _geom_body(vt_ref, wr1_ref, wr2_ref, r_ref, u_ref):
    v = vt_ref[...]
    r2 = jnp.sum(v * v, axis=0, keepdims=True) + 1e-12
    r = jnp.sqrt(r2)
    rinv = 1.0 / r
    uT = v * rinv

    def t8(a):
        return jnp.transpose(a, (1, 0))

    u_ref[...] = t8(uT)
    x = r / RCUT
    nv = lax.broadcasted_iota(jnp.int32, (RBF, 1), 0).astype(_F32) + 1.0
    bes = jnp.sqrt(2.0 / RCUT) * jnp.sin(nv * (jnp.pi * x)) * rinv
    x6 = x * x * x
    x6 = x6 * x6
    fcut = (1.0 - 28.0 * x6 + 48.0 * x6 * x - 21.0 * x6 * x * x)
    fcut = jnp.where(x < 1.0, fcut, 0.0)
    rb = t8(bes * fcut)
    r_ref[...] = _dot(_silu(_dot(rb, wr1_ref[...])), wr2_ref[...])


def _tc_geom(vecT, wr1, wr2):
    return pl.pallas_call(
        _geom_body,
        grid=(EE // BE,),
        in_specs=[pl.BlockSpec((8, BE), lambda i: (0, i)),
                  pl.BlockSpec((RBF, CC), lambda i: (0, 0)),
                  pl.BlockSpec((CC, CC), lambda i: (0, 0))],
        out_specs=[pl.BlockSpec((BE, CC), lambda i: (i, 0)),
                   pl.BlockSpec((BE, 8), lambda i: (i, 0))],
        out_shape=[jax.ShapeDtypeStruct((EE, CC), _F32),
                   jax.ShapeDtypeStruct((EE, 8), _F32)],
    )(vecT, wr1, wr2)


def _msg_body(r_ref, u_ref, hs_ref, m_ref):
    rr = r_ref[...]
    hs = hs_ref[...]
    u = u_ref[...]
    m0 = rr * hs[:, :CC]
    m1 = rr * hs[:, CC:]
    m_ref[0] = jnp.concatenate([m0, m1 * u[:, 0:1]], axis=1)
    m_ref[1] = jnp.concatenate([m1 * u[:, 1:2], m1 * u[:, 2:3]], axis=1)


def _tc_msg(r_arr, u8, hsg):
    return pl.pallas_call(
        _msg_body,
        grid=(EE // BE,),
        in_specs=[pl.BlockSpec((BE, CC), lambda i: (i, 0)),
                  pl.BlockSpec((BE, 8), lambda i: (i, 0)),
                  pl.BlockSpec((BE, 2 * CC), lambda i: (i, 0))],
        out_specs=pl.BlockSpec((2, BE, 2 * CC), lambda i: (0, i, 0)),
        out_shape=jax.ShapeDtypeStruct((2, EE, 2 * CC), _F32),
    )(r_arr, u8, hsg)


def _node_body(a_ref, wu_ref, wm_ref, wv_ref, out_ref):
    a = a_ref[...] * (1.0 / AVGN)
    a1x = a[0, :, CC:]
    a1y = a[1, :, :CC]
    a1z = a[1, :, CC:]
    inv = a[0, :, :CC] + a1x * a1x + a1y * a1y + a1z * a1z
    h = _silu(_dot(inv, wu_ref[...]))
    out_ref[...] = jnp.concatenate(
        [_dot(h, wm_ref[...]), _dot(h, wv_ref[...])], axis=1)


def _tc_node(a_arr, wu, wm, wv):
    return pl.pallas_call(
        _node_body,
        grid=(NN // BN,),
        in_specs=[pl.BlockSpec((2, BN, 2 * CC), lambda i: (0, i, 0)),
                  pl.BlockSpec((CC, CC), lambda i: (0, 0)),
                  pl.BlockSpec((CC, CC), lambda i: (0, 0)),
                  pl.BlockSpec((CC, CC), lambda i: (0, 0))],
        out_specs=pl.BlockSpec((BN, 2 * CC), lambda i: (i, 0)),
        out_shape=jax.ShapeDtypeStruct((NN, 2 * CC), _F32),
    )(a_arr, wu, wm, wv)


def _final_body(a_ref, wu_ref, w1_ref, w2_ref, spec_ref, ae_ref, bat_ref,
                out_ref):
    @pl.when(pl.program_id(0) == 0)
    def _():
        out_ref[...] = jnp.zeros_like(out_ref)

    a = a_ref[...] * (1.0 / AVGN)
    a1x = a[0, :, CC:]
    a1y = a[1, :, :CC]
    a1z = a[1, :, CC:]
    inv = a[0, :, :CC] + a1x * a1x + a1y * a1y + a1z * a1z
    h = _silu(_dot(inv, wu_ref[...]))
    e = _dot(_silu(_dot(h, w1_ref[...])), w2_ref[...])
    oh_s = (spec_ref[...] == lax.broadcasted_iota(jnp.int32, (1, 16), 1))
    e = e + _dot(oh_s.astype(_F32), ae_ref[...])
    oh_b = (bat_ref[...] == lax.broadcasted_iota(jnp.int32, (1, 16), 1))
    out_ref[...] += jnp.sum(oh_b.astype(_F32) * e, axis=0, keepdims=True)


def _tc_final(a_arr, wu, w1, w2, spec, ae16, bat):
    return pl.pallas_call(
        _final_body,
        grid=(NN // BN,),
        in_specs=[pl.BlockSpec((2, BN, 2 * CC), lambda i: (0, i, 0)),
                  pl.BlockSpec((CC, CC), lambda i: (0, 0)),
                  pl.BlockSpec((CC, 16), lambda i: (0, 0)),
                  pl.BlockSpec((16, 1), lambda i: (0, 0)),
                  pl.BlockSpec((BN, 1), lambda i: (i, 0)),
                  pl.BlockSpec((16, 1), lambda i: (0, 0)),
                  pl.BlockSpec((BN, 1), lambda i: (i, 0))],
        out_specs=pl.BlockSpec((1, 16), lambda i: (0, 0)),
        out_shape=jax.ShapeDtypeStruct((1, 16), _F32),
    )(a_arr, wu, w1, w2, spec, ae16, bat)


def kernel(positions, edge_index, species, batch,
           W_embed, W_r1, W_r2,
           W_msg0, W_vec0, W_up0,
           W_msg1, W_vec1, W_up1,
           W_ro1, W_ro2, atomic_E):
    src = edge_index[0].astype(jnp.int32)
    dst = edge_index[1].astype(jnp.int32)
    pos1d = jnp.pad(positions.astype(_F32), ((0, 0), (0, 5))).reshape(-1)
    we16 = jnp.pad(W_embed, ((0, 6), (0, 0)))
    ae16 = jnp.pad(atomic_E, (0, 6)).reshape(16, 1)
    spec = species.astype(jnp.int32).reshape(NN, 1)
    bat = batch.astype(jnp.int32).reshape(NN, 1)
    zeros_acc = jnp.zeros((NN, 2 * CC), _F32)

    hmv = _tc_node0(spec, we16, W_msg0, W_vec0)
    vecT, hsg = _sc_gather0(src, dst, pos1d, hmv)
    r_arr, u8 = _tc_geom(vecT, W_r1, W_r2)

    m = _tc_msg(r_arr, u8, hsg)
    a_arr = _sc_scatter(m, dst, zeros_acc)
    hmv = _tc_node(a_arr, W_up0, W_msg1, W_vec1)

    hsg = _sc_gather1(src, hmv)
    m = _tc_msg(r_arr, u8, hsg)
    a_arr = _sc_scatter(m, dst, zeros_acc)

    energy = _tc_final(a_arr, W_up1, W_ro1, W_ro2, spec, ae16, bat)
    return energy.reshape(NGR)
